# Initial kernel scaffold; baseline (speedup 1.0000x reference)
import jax
import jax.numpy as jnp
from jax.experimental import pallas as pl


def kernel(g_edge_index, h, h_in, W, b, gamma, beta):
    raise NotImplementedError("write your pallas kernel here")


# speedup vs baseline: 7.2196x; 7.2196x over previous
import functools

import jax
import jax.numpy as jnp
from jax import lax
from jax.experimental import pallas as pl
from jax.experimental.pallas import tpu as pltpu
from jax.experimental.pallas import tpu_sc as plsc

N_NODES = 10000
D = 128
DP = 144
NC, NS = 2, 16
NW = NC * NS
E = 320000
EPW = E // NW
B = 100
NCHUNK = EPW // B
ROWS_PER_TILE = N_NODES // NS

BL = 1000
NB = N_NODES // BL


def _sc_segment_sum(src2, dst2, h_aug, zeros_nd):
    mesh = plsc.VectorSubcoreMesh(core_axis_name="c", subcore_axis_name="s")

    @functools.partial(
        pl.kernel,
        out_type=jax.ShapeDtypeStruct((NC * N_NODES, DP), jnp.float32),
        mesh=mesh,
        compiler_params=pltpu.CompilerParams(use_tc_tiling_on_sc=False),
        scratch_types=[
            pltpu.VMEM((NCHUNK, B), jnp.int32),
            pltpu.VMEM((NCHUNK, B), jnp.int32),
            pltpu.VMEM((B, DP), jnp.float32),
            pltpu.VMEM_SHARED((N_NODES, DP), jnp.float32),
        ],
    )
    def k(src_hbm, dst_hbm, h_hbm, z_hbm, out_hbm, src_v, dst_v, rows_v, acc_sh):
        c = lax.axis_index("c")
        s = lax.axis_index("s")
        wid = s * NC + c
        pltpu.sync_copy(
            z_hbm.at[pl.ds(s * ROWS_PER_TILE, ROWS_PER_TILE)],
            acc_sh.at[pl.ds(s * ROWS_PER_TILE, ROWS_PER_TILE)],
        )
        pltpu.sync_copy(src_hbm.at[wid], src_v)
        pltpu.sync_copy(dst_hbm.at[wid], dst_v)
        plsc.subcore_barrier()

        @pl.loop(0, NCHUNK)
        def _(j):
            pltpu.sync_copy(h_hbm.at[src_v.at[j]], rows_v)
            pltpu.sync_copy(rows_v, acc_sh.at[dst_v.at[j]], add=True)

        plsc.subcore_barrier()
        pltpu.sync_copy(
            acc_sh.at[pl.ds(s * ROWS_PER_TILE, ROWS_PER_TILE)],
            out_hbm.at[pl.ds(c * N_NODES + s * ROWS_PER_TILE, ROWS_PER_TILE)],
        )

    return k(src2, dst2, h_aug, zeros_nd)


def _tc_linear_stats(p0, p1, h_in, W, b2):
    def body(p0_ref, p1_ref, hin_ref, w_ref, b_ref, y_ref, sum_ref, sq_ref):
        i = pl.program_id(0)
        tot = p0_ref[...] + p1_ref[...]
        deg = jnp.maximum(tot[:, D : D + 1], 1.0)
        x = tot[:, :D] / deg + hin_ref[...]
        y = (
            lax.dot_general(
                x, w_ref[...], (((1,), (1,)), ((), ())),
                preferred_element_type=jnp.float32,
            )
            + b_ref[...]
        )
        y_ref[...] = y

        @pl.when(i == 0)
        def _():
            sum_ref[...] = jnp.zeros_like(sum_ref)
            sq_ref[...] = jnp.zeros_like(sq_ref)

        sum_ref[...] += jnp.sum(y, axis=0, keepdims=True)
        sq_ref[...] += jnp.sum(y * y, axis=0, keepdims=True)

    return pl.pallas_call(
        body,
        grid=(NB,),
        in_specs=[
            pl.BlockSpec((BL, DP), lambda i: (i, 0)),
            pl.BlockSpec((BL, DP), lambda i: (i, 0)),
            pl.BlockSpec((BL, D), lambda i: (i, 0)),
            pl.BlockSpec((D, D), lambda i: (0, 0)),
            pl.BlockSpec((1, D), lambda i: (0, 0)),
        ],
        out_specs=[
            pl.BlockSpec((BL, D), lambda i: (i, 0)),
            pl.BlockSpec((1, D), lambda i: (0, 0)),
            pl.BlockSpec((1, D), lambda i: (0, 0)),
        ],
        out_shape=[
            jax.ShapeDtypeStruct((N_NODES, D), jnp.float32),
            jax.ShapeDtypeStruct((1, D), jnp.float32),
            jax.ShapeDtypeStruct((1, D), jnp.float32),
        ],
    )(p0, p1, h_in, W, b2)


def _tc_bn_relu(y, ssum, ssq, gamma2, beta2):
    def body(y_ref, sum_ref, sq_ref, g_ref, be_ref, o_ref):
        mean = sum_ref[...] * (1.0 / N_NODES)
        var = sq_ref[...] * (1.0 / N_NODES) - mean * mean
        inv = lax.rsqrt(var + 1e-5)
        o_ref[...] = jnp.maximum(
            (y_ref[...] - mean) * (inv * g_ref[...]) + be_ref[...], 0.0
        )

    return pl.pallas_call(
        body,
        grid=(NB,),
        in_specs=[
            pl.BlockSpec((BL, D), lambda i: (i, 0)),
            pl.BlockSpec((1, D), lambda i: (0, 0)),
            pl.BlockSpec((1, D), lambda i: (0, 0)),
            pl.BlockSpec((1, D), lambda i: (0, 0)),
            pl.BlockSpec((1, D), lambda i: (0, 0)),
        ],
        out_specs=pl.BlockSpec((BL, D), lambda i: (i, 0)),
        out_shape=jax.ShapeDtypeStruct((N_NODES, D), jnp.float32),
    )(y, ssum, ssq, gamma2, beta2)


def kernel(g_edge_index, h, h_in, W, b, gamma, beta):
    ei = g_edge_index.astype(jnp.int32)
    src2 = ei[0].reshape(NW, NCHUNK, B)
    dst2 = ei[1].reshape(NW, NCHUNK, B)
    h_aug = jnp.concatenate(
        [
            h,
            jnp.ones((N_NODES, 1), jnp.float32),
            jnp.zeros((N_NODES, DP - D - 1), jnp.float32),
        ],
        axis=1,
    )
    zeros_nd = jnp.zeros((N_NODES, DP), jnp.float32)
    partials = _sc_segment_sum(src2, dst2, h_aug, zeros_nd)
    p0 = partials[:N_NODES]
    p1 = partials[N_NODES:]
    y, ssum, ssq = _tc_linear_stats(
        p0, p1, h_in, W, b.reshape(1, D)
    )
    return _tc_bn_relu(y, ssum, ssq, gamma.reshape(1, D), beta.reshape(1, D))
